# baseline (speedup 1.0000x reference)
import functools

import jax
import jax.numpy as jnp
from jax import lax
from jax.experimental import pallas as pl
from jax.experimental.pallas import tpu as pltpu
from jax.experimental.pallas import tpu_sc as plsc

VOCAB = 100000
DIM = 64
BATCH = 16384

NC = 2
NS = 16
L = 16
NW = NC * NS
BPW = BATCH // NW
GROUPS = BPW // L

_mesh = plsc.VectorSubcoreMesh(core_axis_name="c", subcore_axis_name="s")


@functools.partial(
    pl.kernel,
    out_type=jax.ShapeDtypeStruct((BATCH,), jnp.float32),
    mesh=_mesh,
    scratch_types=[
        pltpu.VMEM((BPW,), jnp.int32),
        pltpu.VMEM((BPW,), jnp.int32),
        pltpu.VMEM((BPW, DIM), jnp.float32),
        pltpu.VMEM((BPW, DIM), jnp.float32),
        pltpu.VMEM((BPW,), jnp.float32),
        pltpu.SemaphoreType.DMA,
        pltpu.SemaphoreType.DMA,
    ],
    compiler_params=pltpu.CompilerParams(
        needs_layout_passes=False,
        use_tc_tiling_on_sc=False,
    ),
)
def _sc_dot(item_ids_hbm, ctx_ids_hbm, item_tab_hbm, ctx_tab_hbm, out_hbm,
            iidx_v, cidx_v, irows_v, crows_v, out_v, sem_i, sem_c):
    wid = lax.axis_index("s") * NC + lax.axis_index("c")
    base = pl.multiple_of(wid * BPW, BPW)

    pltpu.sync_copy(item_ids_hbm.at[pl.ds(base, BPW)], iidx_v)
    pltpu.sync_copy(ctx_ids_hbm.at[pl.ds(base, BPW)], cidx_v)

    cp_i = pltpu.async_copy(item_tab_hbm.at[iidx_v], irows_v, sem_i)
    cp_c = pltpu.async_copy(ctx_tab_hbm.at[cidx_v], crows_v, sem_c)
    cp_i.wait()
    cp_c.wait()

    def group_body(g, carry):
        row0 = pl.multiple_of(g * L, L)
        rows = row0 + lax.broadcasted_iota(jnp.int32, (L,), 0)
        acc = jnp.zeros((L,), jnp.float32)
        for k in range(DIM):
            col = jnp.full((L,), k, jnp.int32)
            a = plsc.load_gather(irows_v, [rows, col])
            b = plsc.load_gather(crows_v, [rows, col])
            acc = acc + a * b
        out_v[pl.ds(row0, L)] = acc
        return carry

    lax.fori_loop(0, GROUPS, group_body, 0)

    pltpu.sync_copy(out_v, out_hbm.at[pl.ds(base, BPW)])


def kernel(item_ids, context_ids, item_table, context_table):
    return _sc_dot(
        item_ids.astype(jnp.int32),
        context_ids.astype(jnp.int32),
        item_table,
        context_table,
    )


# speedup vs baseline: 1.3174x; 1.3174x over previous
import functools

import jax
import jax.numpy as jnp
from jax import lax
from jax.experimental import pallas as pl
from jax.experimental.pallas import tpu as pltpu
from jax.experimental.pallas import tpu_sc as plsc

VOCAB = 100000
DIM = 64
BATCH = 16384
FUSED = 2 * DIM

NC = 2
NS = 16
L = 16
NW = NC * NS
BPW = BATCH // NW
CH = 256
NCHUNK = BPW // CH
CGROUPS = CH // L

_mesh = plsc.VectorSubcoreMesh(core_axis_name="c", subcore_axis_name="s")


@functools.partial(
    pl.kernel,
    out_type=jax.ShapeDtypeStruct((BATCH,), jnp.float32),
    mesh=_mesh,
    scratch_types=[
        pltpu.VMEM((BPW,), jnp.int32),
        pltpu.VMEM((BPW,), jnp.int32),
        pltpu.VMEM((CH, FUSED), jnp.float32),
        pltpu.VMEM((CH, FUSED), jnp.float32),
        pltpu.VMEM((L * (L + 1),), jnp.float32),
        pltpu.VMEM((BPW,), jnp.float32),
        pltpu.SemaphoreType.DMA,
        pltpu.SemaphoreType.DMA,
    ],
    compiler_params=pltpu.CompilerParams(
        needs_layout_passes=False,
        use_tc_tiling_on_sc=False,
    ),
)
def _sc_dot(item_ids_hbm, ctx_ids_hbm, fused_tab_hbm, out_hbm,
            iidx_v, cidx_v, irows_v, crows_v, tpose_v, out_v, sem_i, sem_c):
    wid = lax.axis_index("s") * NC + lax.axis_index("c")
    base = pl.multiple_of(wid * BPW, BPW)

    pltpu.sync_copy(item_ids_hbm.at[pl.ds(base, BPW)], iidx_v)
    pltpu.sync_copy(ctx_ids_hbm.at[pl.ds(base, BPW)], cidx_v)

    lanes = lax.broadcasted_iota(jnp.int32, (L,), 0)

    for ck in range(NCHUNK):
        cp_i = pltpu.async_copy(
            fused_tab_hbm.at[iidx_v.at[pl.ds(ck * CH, CH)]], irows_v, sem_i
        )
        cp_c = pltpu.async_copy(
            fused_tab_hbm.at[cidx_v.at[pl.ds(ck * CH, CH)]], crows_v, sem_c
        )
        cp_i.wait()
        cp_c.wait()

        def group_body(g, carry):
            row0 = pl.multiple_of(g * L, L)
            for r in range(L):
                row = row0 + r
                acc = jnp.zeros((L,), jnp.float32)
                for c in range(DIM // L):
                    a = irows_v[row, pl.ds(c * L, L)]
                    b = crows_v[row, pl.ds(DIM + c * L, L)]
                    acc = acc + a * b
                plsc.store_scatter(tpose_v, [lanes * (L + 1) + r], acc)
            s = jnp.zeros((L,), jnp.float32)
            for l in range(L):
                s = s + tpose_v[pl.ds(l * (L + 1), L)]
            out_v[pl.ds(ck * CH + row0, L)] = s
            return carry

        lax.fori_loop(0, CGROUPS, group_body, 0)

    pltpu.sync_copy(out_v, out_hbm.at[pl.ds(base, BPW)])


def kernel(item_ids, context_ids, item_table, context_table):
    fused = jnp.concatenate([item_table, context_table], axis=1)
    return _sc_dot(
        item_ids.astype(jnp.int32),
        context_ids.astype(jnp.int32),
        fused,
    )
